# baseline (speedup 1.0000x reference)
import functools

import jax
import jax.numpy as jnp
from jax import lax
from jax.experimental import pallas as pl
from jax.experimental.pallas import tpu as pltpu

E = 8
TOP_K = 2
DIM = 1024
INTER = 512
NV = 10


def _silu(v):
    return v * (1.0 / (1.0 + jnp.exp(-v)))


def _moe_block(x_ref, gw_ref, gb_ref, v1_ref, vb1_ref, v2_ref, vb2_ref,
               v3_ref, vb3_ref, out_ref, comb_ref):
    e = pl.program_id(1)
    xs = x_ref[...]

    @pl.when(e == 0)
    def _gate():
        logits = lax.dot_general(xs, gw_ref[...], (((1,), (1,)), ((), ())),
                                 preferred_element_type=jnp.float32)
        logits = logits + gb_ref[...]
        m = jnp.max(logits, axis=1, keepdims=True)
        ex = jnp.exp(logits - m)
        scores = ex / jnp.sum(ex, axis=1, keepdims=True)
        cols = lax.broadcasted_iota(jnp.int32, scores.shape, 1)
        m1 = jnp.max(scores, axis=1, keepdims=True)
        i1 = jnp.min(jnp.where(scores == m1, cols, E), axis=1, keepdims=True)
        masked = jnp.where(cols == i1, -jnp.inf, scores)
        m2 = jnp.max(masked, axis=1, keepdims=True)
        i2 = jnp.min(jnp.where(masked == m2, cols, E), axis=1, keepdims=True)
        denom = m1 + m2 + 1e-20
        w1 = m1 / denom
        w2 = m2 / denom
        cols16 = lax.broadcasted_iota(jnp.int32, (xs.shape[0], 16), 1)
        comb = (jnp.where(cols16 == i1, w1, 0.0)
                + jnp.where(cols16 == i2, w2, 0.0)
                + jnp.where(cols16 >= E, 1.0, 0.0))
        comb_ref[...] = comb

    xb = xs.astype(jnp.bfloat16)
    h1 = lax.dot_general(xb, v1_ref[0].astype(jnp.bfloat16),
                         (((1,), (1,)), ((), ())),
                         preferred_element_type=jnp.float32) + vb1_ref[0]
    h3 = lax.dot_general(xb, v3_ref[0].astype(jnp.bfloat16),
                         (((1,), (1,)), ((), ())),
                         preferred_element_type=jnp.float32) + vb3_ref[0]
    h = _silu(h1) * h3
    oe = lax.dot_general(h.astype(jnp.bfloat16), v2_ref[0].astype(jnp.bfloat16),
                         (((1,), (1,)), ((), ())),
                         preferred_element_type=jnp.float32) + vb2_ref[0]
    cols16 = lax.broadcasted_iota(jnp.int32, comb_ref.shape, 1)
    ce = jnp.sum(jnp.where(cols16 == e, comb_ref[...], 0.0), axis=1,
                 keepdims=True)

    @pl.when(e == 0)
    def _init():
        out_ref[...] = oe * ce

    @pl.when(e != 0)
    def _acc():
        out_ref[...] = out_ref[...] + oe * ce


@functools.partial(jax.jit, static_argnames=("bm",))
def _moe_dense(xf, gate_w, gate_b, V1, VB1, V2, VB2, V3, VB3, bm=1024):
    t = xf.shape[0]
    grid = (t // bm, NV)
    return pl.pallas_call(
        _moe_block,
        grid=grid,
        in_specs=[
            pl.BlockSpec((bm, DIM), lambda i, e: (i, 0)),
            pl.BlockSpec((E, DIM), lambda i, e: (0, 0)),
            pl.BlockSpec((1, E), lambda i, e: (0, 0)),
            pl.BlockSpec((1, INTER, DIM), lambda i, e: (e, 0, 0)),
            pl.BlockSpec((1, 1, INTER), lambda i, e: (e, 0, 0)),
            pl.BlockSpec((1, DIM, INTER), lambda i, e: (e, 0, 0)),
            pl.BlockSpec((1, 1, DIM), lambda i, e: (e, 0, 0)),
            pl.BlockSpec((1, INTER, DIM), lambda i, e: (e, 0, 0)),
            pl.BlockSpec((1, 1, INTER), lambda i, e: (e, 0, 0)),
        ],
        out_specs=pl.BlockSpec((bm, DIM), lambda i, e: (i, 0)),
        out_shape=jax.ShapeDtypeStruct((t, DIM), jnp.float32),
        scratch_shapes=[pltpu.VMEM((bm, 16), jnp.float32)],
        compiler_params=pltpu.CompilerParams(
            dimension_semantics=("arbitrary", "arbitrary")),
    )(xf, gate_w, gate_b.reshape(1, E),
      V1, VB1[:, None], V2, VB2[:, None], V3, VB3[:, None])


def kernel(x, gate_w, gate_b, W1, B1, W2, B2, W3, B3, SW1, SB1, SW2, SB2, SW3, SB3):
    bsz, seq, h = x.shape
    xf = x.reshape(-1, h)
    V1 = jnp.concatenate([W1, SW1.reshape(2, INTER, DIM)], axis=0)
    V3 = jnp.concatenate([W3, SW3.reshape(2, INTER, DIM)], axis=0)
    V2 = jnp.concatenate(
        [W2, SW2.reshape(DIM, 2, INTER).transpose(1, 0, 2)], axis=0)
    VB1 = jnp.concatenate([B1, SB1.reshape(2, INTER)], axis=0)
    VB3 = jnp.concatenate([B3, SB3.reshape(2, INTER)], axis=0)
    VB2 = jnp.concatenate(
        [B2, SB2[None], jnp.zeros((1, DIM), jnp.float32)], axis=0)
    y = _moe_dense(xf, gate_w, gate_b, V1, VB1, V2, VB2, V3, VB3)
    return y.reshape(bsz, seq, h)


# speedup vs baseline: 1.1013x; 1.1013x over previous
import functools

import jax
import jax.numpy as jnp
from jax import lax
from jax.experimental import pallas as pl
from jax.experimental.pallas import tpu as pltpu
from jax.experimental.pallas import tpu_sc as plsc

E = 8
DIM = 1024
INTER = 512
SINTER = 1024
T = 4096
TS = 2 * T
BLK = 256
NB = TS // BLK
NT = NB + E - 1
NW = 32
CHUNK = TS // NW
TPW = T // NW
SUB = 4
SUBN = CHUNK // SUB
TSUB = TPW // SUB


def _m2i(mask):
    return jnp.where(mask, jnp.ones((16,), jnp.int32),
                     jnp.zeros((16,), jnp.int32))


def _silu(v):
    return v * (1.0 / (1.0 + jnp.exp(-v)))


def _gate_block(x_ref, gw_ref, gb_ref, e_ref, w1_ref, w2_ref):
    xs = x_ref[...]
    logits = lax.dot_general(xs, gw_ref[...], (((1,), (1,)), ((), ())),
                             preferred_element_type=jnp.float32)
    logits = logits + gb_ref[...]
    m = jnp.max(logits, axis=1, keepdims=True)
    ex = jnp.exp(logits - m)
    scores = ex / jnp.sum(ex, axis=1, keepdims=True)
    cols = lax.broadcasted_iota(jnp.int32, scores.shape, 1)
    m1 = jnp.max(scores, axis=1, keepdims=True)
    i1 = jnp.min(jnp.where(scores == m1, cols, E), axis=1, keepdims=True)
    masked = jnp.where(cols == i1, -jnp.inf, scores)
    m2 = jnp.max(masked, axis=1, keepdims=True)
    i2 = jnp.min(jnp.where(masked == m2, cols, E), axis=1, keepdims=True)
    denom = m1 + m2 + 1e-20
    e_ref[...] = jnp.concatenate([i1, i2], axis=1)
    w1_ref[...] = jnp.broadcast_to(m1 / denom, w1_ref.shape)
    w2_ref[...] = jnp.broadcast_to(m2 / denom, w2_ref.shape)


def _gate(xf, gate_w, gate_b):
    bm = 1024
    return pl.pallas_call(
        _gate_block,
        grid=(T // bm,),
        in_specs=[
            pl.BlockSpec((bm, DIM), lambda i: (i, 0)),
            pl.BlockSpec((E, DIM), lambda i: (0, 0)),
            pl.BlockSpec((1, E), lambda i: (0, 0)),
        ],
        out_specs=[
            pl.BlockSpec((bm, 2), lambda i: (i, 0)),
            pl.BlockSpec((bm, 128), lambda i: (i, 0)),
            pl.BlockSpec((bm, 128), lambda i: (i, 0)),
        ],
        out_shape=[
            jax.ShapeDtypeStruct((T, 2), jnp.int32),
            jax.ShapeDtypeStruct((T, 128), jnp.float32),
            jax.ShapeDtypeStruct((T, 128), jnp.float32),
        ],
    )(xf, gate_w, gate_b.reshape(1, E))


def _b1_block(e_ref, dest_ref, cnt_ref):
    ev = e_ref[...]
    r128 = lax.broadcasted_iota(jnp.int32, (128, 128), 0)
    c128 = lax.broadcasted_iota(jnp.int32, (128, 128), 1)
    su = jnp.where(r128 < c128, 1.0, 0.0)
    r64 = lax.broadcasted_iota(jnp.int32, (64, 64), 0)
    c64 = lax.broadcasted_iota(jnp.int32, (64, 64), 1)
    sl = jnp.where(c64 < r64, 1.0, 0.0)
    dest = jnp.zeros(ev.shape, jnp.float32)
    cnts = jnp.zeros((1, 16), jnp.float32)
    lane16 = lax.broadcasted_iota(jnp.int32, (1, 16), 1)
    start = 0.0
    for e in range(E):
        m = jnp.where(ev == e, 1.0, 0.0)
        pw = lax.dot_general(m, su, (((1,), (0,)), ((), ())),
                             preferred_element_type=jnp.float32)
        s = jnp.sum(m, axis=1, keepdims=True)
        o = lax.dot_general(sl, s, (((1,), (0,)), ((), ())),
                            preferred_element_type=jnp.float32)
        rank = pw + o
        total = jnp.sum(s)
        dest = dest + m * (start + rank)
        cnts = cnts + jnp.where(lane16 == e, total, 0.0)
        start = start + total
    dest_ref[...] = dest.astype(jnp.int32)
    cnt_ref[...] = cnts.astype(jnp.int32)


def _b1(eperm):
    dest2d, cnt2d = pl.pallas_call(
        _b1_block,
        grid=(1,),
        in_specs=[pl.BlockSpec((64, 128), lambda i: (0, 0))],
        out_specs=[
            pl.BlockSpec((64, 128), lambda i: (0, 0)),
            pl.BlockSpec((1, 16), lambda i: (0, 0)),
        ],
        out_shape=[
            jax.ShapeDtypeStruct((64, 128), jnp.int32),
            jax.ShapeDtypeStruct((1, 16), jnp.int32),
        ],
    )(eperm.reshape(64, 128))
    return dest2d.reshape(TS), cnt2d.reshape(16)


def _b2_body(xhbm, wperm, dest, sortedx, wsorted, didx, idxv, rows, wbuf,
             sem):
    wid = lax.axis_index("s") * 2 + lax.axis_index("c")
    base = wid * CHUNK
    lane = lax.broadcasted_iota(jnp.int32, (16,), 0)

    for c in range(SUB):
        pltpu.sync_copy(dest.at[pl.ds(base + SUBN * c, SUBN)], didx)
        tb = wid * TPW + (SUBN * c) % TPW
        for k in range(SUBN // 16):
            idxv[pl.ds(16 * k, 16)] = lane + (tb + 16 * k)
        pltpu.async_copy(xhbm.at[idxv], rows, sem).wait()
        pltpu.async_copy(rows, sortedx.at[didx], sem).wait()
        pltpu.sync_copy(wperm.at[pl.ds(base + SUBN * c, SUBN)], wbuf)
        pltpu.async_copy(wbuf, wsorted.at[didx], sem).wait()


def _b2(xf, wperm, dest):
    mesh = plsc.VectorSubcoreMesh(core_axis_name="c", subcore_axis_name="s")
    f = pl.kernel(
        _b2_body,
        out_type=[
            jax.ShapeDtypeStruct((TS, DIM), jnp.float32),
            jax.ShapeDtypeStruct((TS, 128), jnp.float32),
        ],
        mesh=mesh,
        scratch_types=[
            pltpu.VMEM((SUBN,), jnp.int32),
            pltpu.VMEM((SUBN,), jnp.int32),
            pltpu.VMEM((SUBN, DIM), jnp.float32),
            pltpu.VMEM((SUBN, 128), jnp.float32),
            pltpu.SemaphoreType.DMA,
        ],
    )
    return f(xf, wperm, dest)


def _metadata(counts16):
    cnt = counts16[:E]
    offs = jnp.concatenate(
        [jnp.zeros((1,), jnp.int32), jnp.cumsum(cnt, dtype=jnp.int32)])
    first_blk = offs[:E] // BLK
    last_blk = jnp.maximum(offs[1:] - 1, 0) // BLK
    nt_e = jnp.where(cnt > 0, last_blk - first_blk + 1, 0).astype(jnp.int32)
    tstart = jnp.concatenate(
        [jnp.zeros((1,), jnp.int32), jnp.cumsum(nt_e, dtype=jnp.int32)])
    ntact = tstart[E]
    tau = jnp.arange(NT, dtype=jnp.int32)
    eid = jnp.sum((tau[:, None] >= tstart[None, :E]).astype(jnp.int32),
                  axis=1) - 1
    eid = jnp.clip(eid, 0, E - 1)
    valid = tau < ntact
    blk = first_blk[eid] + (tau - tstart[eid])
    blk = jnp.where(valid, blk, NB - 1)
    gs = jnp.where(valid, offs[eid], TS)
    gend = jnp.where(valid, offs[eid + 1], TS)
    eid = jnp.where(valid, eid, E - 1)
    ini = jnp.concatenate([jnp.ones((1,), jnp.int32),
                           (blk[1:] != blk[:-1]).astype(jnp.int32)])
    ini = ini * valid.astype(jnp.int32)
    return eid, blk, gs, gend, ini


def _grouped_block(eid_r, blk_r, gs_r, ge_r, ini_r, xs_ref, ws_ref, w1_ref,
                   vb1_ref, w2_ref, vb2_ref, w3_ref, vb3_ref, out_ref):
    t = pl.program_id(0)
    xs = xs_ref[...]
    h1 = lax.dot_general(xs, w1_ref[0], (((1,), (1,)), ((), ())),
                         preferred_element_type=jnp.float32) + vb1_ref[0]
    h3 = lax.dot_general(xs, w3_ref[0], (((1,), (1,)), ((), ())),
                         preferred_element_type=jnp.float32) + vb3_ref[0]
    h = _silu(h1) * h3
    oe = lax.dot_general(h, w2_ref[0], (((1,), (1,)), ((), ())),
                         preferred_element_type=jnp.float32) + vb2_ref[0]
    oe = oe * ws_ref[:, 0:1]
    rows = blk_r[t] * BLK + lax.broadcasted_iota(jnp.int32, (BLK, 1), 0)
    active = (rows >= gs_r[t]) & (rows < ge_r[t])
    oe = jnp.where(active, oe, 0.0)

    @pl.when(ini_r[t] == 1)
    def _():
        out_ref[...] = oe

    @pl.when(ini_r[t] == 0)
    def _():
        out_ref[...] = out_ref[...] + oe


def _grouped(sorted_x, wsorted, W1, B1, W2, B2, W3, B3, eid, blk, gs, gend,
             ini):
    grid_spec = pltpu.PrefetchScalarGridSpec(
        num_scalar_prefetch=5,
        grid=(NT,),
        in_specs=[
            pl.BlockSpec((BLK, DIM), lambda t, ei, bl, g0, g1, i0: (bl[t], 0)),
            pl.BlockSpec((BLK, 128), lambda t, ei, bl, g0, g1, i0: (bl[t], 0)),
            pl.BlockSpec((1, INTER, DIM),
                         lambda t, ei, bl, g0, g1, i0: (ei[t], 0, 0)),
            pl.BlockSpec((1, 1, INTER),
                         lambda t, ei, bl, g0, g1, i0: (ei[t], 0, 0)),
            pl.BlockSpec((1, DIM, INTER),
                         lambda t, ei, bl, g0, g1, i0: (ei[t], 0, 0)),
            pl.BlockSpec((1, 1, DIM),
                         lambda t, ei, bl, g0, g1, i0: (ei[t], 0, 0)),
            pl.BlockSpec((1, INTER, DIM),
                         lambda t, ei, bl, g0, g1, i0: (ei[t], 0, 0)),
            pl.BlockSpec((1, 1, INTER),
                         lambda t, ei, bl, g0, g1, i0: (ei[t], 0, 0)),
        ],
        out_specs=pl.BlockSpec((BLK, DIM),
                               lambda t, ei, bl, g0, g1, i0: (bl[t], 0)),
    )
    return pl.pallas_call(
        _grouped_block,
        grid_spec=grid_spec,
        out_shape=jax.ShapeDtypeStruct((TS, DIM), jnp.float32),
        compiler_params=pltpu.CompilerParams(
            dimension_semantics=("arbitrary",)),
    )(eid, blk, gs, gend, ini, sorted_x, wsorted, W1, B1[:, None], W2,
      B2[:, None], W3, B3[:, None])


def _shared_block(x_ref, w1_ref, b1_ref, w2_ref, b2_ref, w3_ref, b3_ref,
                  out_ref):
    xs = x_ref[...]
    h1 = lax.dot_general(xs, w1_ref[...], (((1,), (1,)), ((), ())),
                         preferred_element_type=jnp.float32) + b1_ref[...]
    h3 = lax.dot_general(xs, w3_ref[...], (((1,), (1,)), ((), ())),
                         preferred_element_type=jnp.float32) + b3_ref[...]
    h = _silu(h1) * h3
    out_ref[...] = lax.dot_general(h, w2_ref[...], (((1,), (1,)), ((), ())),
                                   preferred_element_type=jnp.float32
                                   ) + b2_ref[...]


def _shared(xf, SW1, SB1, SW2, SB2, SW3, SB3):
    bm = 1024
    return pl.pallas_call(
        _shared_block,
        grid=(T // bm,),
        in_specs=[
            pl.BlockSpec((bm, DIM), lambda i: (i, 0)),
            pl.BlockSpec((SINTER, DIM), lambda i: (0, 0)),
            pl.BlockSpec((1, SINTER), lambda i: (0, 0)),
            pl.BlockSpec((DIM, SINTER), lambda i: (0, 0)),
            pl.BlockSpec((1, DIM), lambda i: (0, 0)),
            pl.BlockSpec((SINTER, DIM), lambda i: (0, 0)),
            pl.BlockSpec((1, SINTER), lambda i: (0, 0)),
        ],
        out_specs=pl.BlockSpec((bm, DIM), lambda i: (i, 0)),
        out_shape=jax.ShapeDtypeStruct((T, DIM), jnp.float32),
    )(xf, SW1, SB1.reshape(1, SINTER), SW2, SB2.reshape(1, DIM),
      SW3, SB3.reshape(1, SINTER))


def _combine_body(shared_hbm, routed_hbm, dest, yhbm, idx0, idx1, r0buf,
                  r1buf, sbuf, sem):
    wid = lax.axis_index("s") * 2 + lax.axis_index("c")
    base = wid * CHUNK

    pltpu.sync_copy(dest.at[pl.ds(base, CHUNK)], idx0)
    for c in range(SUB):
        tbase = wid * TPW + TSUB * c
        pltpu.async_copy(
            routed_hbm.at[idx0.at[pl.ds(TSUB * c, TSUB)]], r0buf, sem).wait()
        pltpu.async_copy(
            routed_hbm.at[idx0.at[pl.ds(TPW + TSUB * c, TSUB)]], r1buf,
            sem).wait()
        pltpu.sync_copy(shared_hbm.at[pl.ds(tbase, TSUB)], sbuf)

        def body(t, carry):
            for d in range(DIM // 16):
                sl = pl.ds(16 * d, 16)
                sbuf[t, sl] = sbuf[t, sl] + r0buf[t, sl] + r1buf[t, sl]
            return carry

        lax.fori_loop(0, TSUB, body, 0)
        pltpu.sync_copy(sbuf, yhbm.at[pl.ds(tbase, TSUB)])


def _combine(shared, routed, dest):
    mesh = plsc.VectorSubcoreMesh(core_axis_name="c", subcore_axis_name="s")
    f = pl.kernel(
        _combine_body,
        out_type=jax.ShapeDtypeStruct((T, DIM), jnp.float32),
        mesh=mesh,
        scratch_types=[
            pltpu.VMEM((CHUNK,), jnp.int32),
            pltpu.VMEM((TSUB,), jnp.int32),
            pltpu.VMEM((TSUB, DIM), jnp.float32),
            pltpu.VMEM((TSUB, DIM), jnp.float32),
            pltpu.VMEM((TSUB, DIM), jnp.float32),
            pltpu.SemaphoreType.DMA,
        ],
    )
    return f(shared, routed, dest)


def kernel(x, gate_w, gate_b, W1, B1, W2, B2, W3, B3, SW1, SB1, SW2, SB2, SW3, SB3):
    bsz, seq, hdim = x.shape
    xf = x.reshape(-1, hdim)
    e2d, w116, w216 = _gate(xf, gate_w, gate_b)
    eperm = e2d.reshape(NW, TPW, 2).transpose(0, 2, 1).reshape(TS)
    wperm = jnp.stack([w116.reshape(NW, TPW, 128), w216.reshape(NW, TPW, 128)],
                      axis=1).reshape(TS, 128)
    dest, counts16 = _b1(eperm)
    sorted_x, wsorted = _b2(xf, wperm, dest)
    eid, blk, gs, gend, ini = _metadata(counts16)
    routed = _grouped(sorted_x, wsorted, W1, B1, W2, B2, W3, B3, eid, blk,
                      gs, gend, ini)
    shared = _shared(xf, SW1, SB1, SW2, SB2, SW3, SB3)
    y = _combine(shared, routed, dest)
    return y.reshape(bsz, seq, hdim)


# speedup vs baseline: 1.1043x; 1.0027x over previous
import functools

import jax
import jax.numpy as jnp
from jax import lax
from jax.experimental import pallas as pl
from jax.experimental.pallas import tpu as pltpu
from jax.experimental.pallas import tpu_sc as plsc

E = 8
DIM = 1024
INTER = 512
SINTER = 1024
T = 4096
TS = 2 * T
BLK = 256
NB = TS // BLK
NT = NB + E - 1
NW = 32
CHUNK = TS // NW
TPW = T // NW
SUB = 4
SUBN = CHUNK // SUB
TSUB = TPW // SUB


def _m2i(mask):
    return jnp.where(mask, jnp.ones((16,), jnp.int32),
                     jnp.zeros((16,), jnp.int32))


def _silu(v):
    return v * (1.0 / (1.0 + jnp.exp(-v)))


def _gate_block(x_ref, gw_ref, gb_ref, e_ref, w1_ref, w2_ref):
    xs = x_ref[...]
    logits = lax.dot_general(xs, gw_ref[...], (((1,), (1,)), ((), ())),
                             preferred_element_type=jnp.float32)
    logits = logits + gb_ref[...]
    m = jnp.max(logits, axis=1, keepdims=True)
    ex = jnp.exp(logits - m)
    scores = ex / jnp.sum(ex, axis=1, keepdims=True)
    cols = lax.broadcasted_iota(jnp.int32, scores.shape, 1)
    m1 = jnp.max(scores, axis=1, keepdims=True)
    i1 = jnp.min(jnp.where(scores == m1, cols, E), axis=1, keepdims=True)
    masked = jnp.where(cols == i1, -jnp.inf, scores)
    m2 = jnp.max(masked, axis=1, keepdims=True)
    i2 = jnp.min(jnp.where(masked == m2, cols, E), axis=1, keepdims=True)
    denom = m1 + m2 + 1e-20
    e_ref[...] = jnp.concatenate([i1, i2], axis=1)
    w1_ref[...] = jnp.broadcast_to(m1 / denom, w1_ref.shape)
    w2_ref[...] = jnp.broadcast_to(m2 / denom, w2_ref.shape)


def _gate(xf, gate_w, gate_b):
    bm = 1024
    return pl.pallas_call(
        _gate_block,
        grid=(T // bm,),
        in_specs=[
            pl.BlockSpec((bm, DIM), lambda i: (i, 0)),
            pl.BlockSpec((E, DIM), lambda i: (0, 0)),
            pl.BlockSpec((1, E), lambda i: (0, 0)),
        ],
        out_specs=[
            pl.BlockSpec((bm, 2), lambda i: (i, 0)),
            pl.BlockSpec((bm, 128), lambda i: (i, 0)),
            pl.BlockSpec((bm, 128), lambda i: (i, 0)),
        ],
        out_shape=[
            jax.ShapeDtypeStruct((T, 2), jnp.int32),
            jax.ShapeDtypeStruct((T, 128), jnp.float32),
            jax.ShapeDtypeStruct((T, 128), jnp.float32),
        ],
    )(xf, gate_w, gate_b.reshape(1, E))


def _b1_block(e_ref, dest_ref, cnt_ref):
    ev = e_ref[...]
    r128 = lax.broadcasted_iota(jnp.int32, (128, 128), 0)
    c128 = lax.broadcasted_iota(jnp.int32, (128, 128), 1)
    su = jnp.where(r128 < c128, 1.0, 0.0)
    r64 = lax.broadcasted_iota(jnp.int32, (64, 64), 0)
    c64 = lax.broadcasted_iota(jnp.int32, (64, 64), 1)
    sl = jnp.where(c64 < r64, 1.0, 0.0)
    dest = jnp.zeros(ev.shape, jnp.float32)
    cnts = jnp.zeros((1, 16), jnp.float32)
    lane16 = lax.broadcasted_iota(jnp.int32, (1, 16), 1)
    start = 0.0
    for e in range(E):
        m = jnp.where(ev == e, 1.0, 0.0)
        pw = lax.dot_general(m, su, (((1,), (0,)), ((), ())),
                             preferred_element_type=jnp.float32)
        s = jnp.sum(m, axis=1, keepdims=True)
        o = lax.dot_general(sl, s, (((1,), (0,)), ((), ())),
                            preferred_element_type=jnp.float32)
        rank = pw + o
        total = jnp.sum(s)
        dest = dest + m * (start + rank)
        cnts = cnts + jnp.where(lane16 == e, total, 0.0)
        start = start + total
    dest_ref[...] = dest.astype(jnp.int32)
    cnt_ref[...] = cnts.astype(jnp.int32)


def _b1(eperm):
    dest2d, cnt2d = pl.pallas_call(
        _b1_block,
        grid=(1,),
        in_specs=[pl.BlockSpec((64, 128), lambda i: (0, 0))],
        out_specs=[
            pl.BlockSpec((64, 128), lambda i: (0, 0)),
            pl.BlockSpec((1, 16), lambda i: (0, 0)),
        ],
        out_shape=[
            jax.ShapeDtypeStruct((64, 128), jnp.int32),
            jax.ShapeDtypeStruct((1, 16), jnp.int32),
        ],
    )(eperm.reshape(64, 128))
    return dest2d.reshape(TS), cnt2d.reshape(16)


B2SUB = 8
B2N = CHUNK // B2SUB


def _b2_body(xhbm, wperm, dest, sortedx, wsorted, didx0, didx1, idxv0, idxv1,
             rows0, rows1, wbuf0, wbuf1, semg0, semg1, sems0, sems1, semw0,
             semw1):
    wid = lax.axis_index("s") * 2 + lax.axis_index("c")
    base = wid * CHUNK
    lane = lax.broadcasted_iota(jnp.int32, (16,), 0)
    didx = [didx0, didx1]
    idxv = [idxv0, idxv1]
    rows = [rows0, rows1]
    wbuf = [wbuf0, wbuf1]
    semg = [semg0, semg1]
    sems = [sems0, sems1]
    semw = [semw0, semw1]
    pend_s = [None, None]
    pend_w = [None, None]

    for c in range(B2SUB):
        b = c & 1
        if pend_s[b] is not None:
            pend_s[b].wait()
            pend_w[b].wait()
        pltpu.sync_copy(dest.at[pl.ds(base + B2N * c, B2N)], didx[b])
        tb = wid * TPW + (B2N * c) % TPW
        for k in range(B2N // 16):
            idxv[b][pl.ds(16 * k, 16)] = lane + (tb + 16 * k)
        g = pltpu.async_copy(xhbm.at[idxv[b]], rows[b], semg[b])
        pltpu.sync_copy(wperm.at[pl.ds(base + B2N * c, B2N)], wbuf[b])
        g.wait()
        pend_s[b] = pltpu.async_copy(rows[b], sortedx.at[didx[b]], sems[b])
        pend_w[b] = pltpu.async_copy(wbuf[b], wsorted.at[didx[b]], semw[b])
    pend_s[0].wait()
    pend_w[0].wait()
    pend_s[1].wait()
    pend_w[1].wait()


def _b2(xf, wperm, dest):
    mesh = plsc.VectorSubcoreMesh(core_axis_name="c", subcore_axis_name="s")
    f = pl.kernel(
        _b2_body,
        out_type=[
            jax.ShapeDtypeStruct((TS, DIM), jnp.float32),
            jax.ShapeDtypeStruct((TS, 128), jnp.float32),
        ],
        mesh=mesh,
        scratch_types=[
            pltpu.VMEM((B2N,), jnp.int32),
            pltpu.VMEM((B2N,), jnp.int32),
            pltpu.VMEM((B2N,), jnp.int32),
            pltpu.VMEM((B2N,), jnp.int32),
            pltpu.VMEM((B2N, DIM), jnp.float32),
            pltpu.VMEM((B2N, DIM), jnp.float32),
            pltpu.VMEM((B2N, 128), jnp.float32),
            pltpu.VMEM((B2N, 128), jnp.float32),
            pltpu.SemaphoreType.DMA,
            pltpu.SemaphoreType.DMA,
            pltpu.SemaphoreType.DMA,
            pltpu.SemaphoreType.DMA,
            pltpu.SemaphoreType.DMA,
            pltpu.SemaphoreType.DMA,
        ],
    )
    return f(xf, wperm, dest)


def _metadata(counts16):
    cnt = counts16[:E]
    offs = jnp.concatenate(
        [jnp.zeros((1,), jnp.int32), jnp.cumsum(cnt, dtype=jnp.int32)])
    first_blk = offs[:E] // BLK
    last_blk = jnp.maximum(offs[1:] - 1, 0) // BLK
    nt_e = jnp.where(cnt > 0, last_blk - first_blk + 1, 0).astype(jnp.int32)
    tstart = jnp.concatenate(
        [jnp.zeros((1,), jnp.int32), jnp.cumsum(nt_e, dtype=jnp.int32)])
    ntact = tstart[E]
    tau = jnp.arange(NT, dtype=jnp.int32)
    eid = jnp.sum((tau[:, None] >= tstart[None, :E]).astype(jnp.int32),
                  axis=1) - 1
    eid = jnp.clip(eid, 0, E - 1)
    valid = tau < ntact
    blk = first_blk[eid] + (tau - tstart[eid])
    blk = jnp.where(valid, blk, NB - 1)
    gs = jnp.where(valid, offs[eid], TS)
    gend = jnp.where(valid, offs[eid + 1], TS)
    eid = jnp.where(valid, eid, E - 1)
    ini = jnp.concatenate([jnp.ones((1,), jnp.int32),
                           (blk[1:] != blk[:-1]).astype(jnp.int32)])
    ini = ini * valid.astype(jnp.int32)
    return eid, blk, gs, gend, ini


def _grouped_block(eid_r, blk_r, gs_r, ge_r, ini_r, xs_ref, ws_ref, w1_ref,
                   vb1_ref, w2_ref, vb2_ref, w3_ref, vb3_ref, out_ref):
    t = pl.program_id(0)
    xs = xs_ref[...]
    h1 = lax.dot_general(xs, w1_ref[0], (((1,), (1,)), ((), ())),
                         preferred_element_type=jnp.float32) + vb1_ref[0]
    h3 = lax.dot_general(xs, w3_ref[0], (((1,), (1,)), ((), ())),
                         preferred_element_type=jnp.float32) + vb3_ref[0]
    h = _silu(h1) * h3
    oe = lax.dot_general(h, w2_ref[0], (((1,), (1,)), ((), ())),
                         preferred_element_type=jnp.float32) + vb2_ref[0]
    oe = oe * ws_ref[:, 0:1]
    rows = blk_r[t] * BLK + lax.broadcasted_iota(jnp.int32, (BLK, 1), 0)
    active = (rows >= gs_r[t]) & (rows < ge_r[t])
    oe = jnp.where(active, oe, 0.0)

    @pl.when(ini_r[t] == 1)
    def _():
        out_ref[...] = oe

    @pl.when(ini_r[t] == 0)
    def _():
        out_ref[...] = out_ref[...] + oe


def _grouped(sorted_x, wsorted, W1, B1, W2, B2, W3, B3, eid, blk, gs, gend,
             ini):
    grid_spec = pltpu.PrefetchScalarGridSpec(
        num_scalar_prefetch=5,
        grid=(NT,),
        in_specs=[
            pl.BlockSpec((BLK, DIM), lambda t, ei, bl, g0, g1, i0: (bl[t], 0)),
            pl.BlockSpec((BLK, 128), lambda t, ei, bl, g0, g1, i0: (bl[t], 0)),
            pl.BlockSpec((1, INTER, DIM),
                         lambda t, ei, bl, g0, g1, i0: (ei[t], 0, 0)),
            pl.BlockSpec((1, 1, INTER),
                         lambda t, ei, bl, g0, g1, i0: (ei[t], 0, 0)),
            pl.BlockSpec((1, DIM, INTER),
                         lambda t, ei, bl, g0, g1, i0: (ei[t], 0, 0)),
            pl.BlockSpec((1, 1, DIM),
                         lambda t, ei, bl, g0, g1, i0: (ei[t], 0, 0)),
            pl.BlockSpec((1, INTER, DIM),
                         lambda t, ei, bl, g0, g1, i0: (ei[t], 0, 0)),
            pl.BlockSpec((1, 1, INTER),
                         lambda t, ei, bl, g0, g1, i0: (ei[t], 0, 0)),
        ],
        out_specs=pl.BlockSpec((BLK, DIM),
                               lambda t, ei, bl, g0, g1, i0: (bl[t], 0)),
    )
    return pl.pallas_call(
        _grouped_block,
        grid_spec=grid_spec,
        out_shape=jax.ShapeDtypeStruct((TS, DIM), jnp.float32),
        compiler_params=pltpu.CompilerParams(
            dimension_semantics=("arbitrary",)),
    )(eid, blk, gs, gend, ini, sorted_x, wsorted, W1, B1[:, None], W2,
      B2[:, None], W3, B3[:, None])


def _shared_block(x_ref, w1_ref, b1_ref, w2_ref, b2_ref, w3_ref, b3_ref,
                  out_ref):
    xs = x_ref[...]
    h1 = lax.dot_general(xs, w1_ref[...], (((1,), (1,)), ((), ())),
                         preferred_element_type=jnp.float32) + b1_ref[...]
    h3 = lax.dot_general(xs, w3_ref[...], (((1,), (1,)), ((), ())),
                         preferred_element_type=jnp.float32) + b3_ref[...]
    h = _silu(h1) * h3
    out_ref[...] = lax.dot_general(h, w2_ref[...], (((1,), (1,)), ((), ())),
                                   preferred_element_type=jnp.float32
                                   ) + b2_ref[...]


def _shared(xf, SW1, SB1, SW2, SB2, SW3, SB3):
    bm = 1024
    return pl.pallas_call(
        _shared_block,
        grid=(T // bm,),
        in_specs=[
            pl.BlockSpec((bm, DIM), lambda i: (i, 0)),
            pl.BlockSpec((SINTER, DIM), lambda i: (0, 0)),
            pl.BlockSpec((1, SINTER), lambda i: (0, 0)),
            pl.BlockSpec((DIM, SINTER), lambda i: (0, 0)),
            pl.BlockSpec((1, DIM), lambda i: (0, 0)),
            pl.BlockSpec((SINTER, DIM), lambda i: (0, 0)),
            pl.BlockSpec((1, SINTER), lambda i: (0, 0)),
        ],
        out_specs=pl.BlockSpec((bm, DIM), lambda i: (i, 0)),
        out_shape=jax.ShapeDtypeStruct((T, DIM), jnp.float32),
    )(xf, SW1, SB1.reshape(1, SINTER), SW2, SB2.reshape(1, DIM),
      SW3, SB3.reshape(1, SINTER))


def _combine_body(shared_hbm, routed_hbm, dest, yhbm, idx0, r0buf,
                  r1buf, sbuf, semg0, semg1, semg2, semy):
    wid = lax.axis_index("s") * 2 + lax.axis_index("c")
    base = wid * CHUNK

    pltpu.sync_copy(dest.at[pl.ds(base, CHUNK)], idx0)
    pend_y = None
    for c in range(SUB):
        tbase = wid * TPW + TSUB * c
        g0 = pltpu.async_copy(
            routed_hbm.at[idx0.at[pl.ds(TSUB * c, TSUB)]], r0buf, semg0)
        g1 = pltpu.async_copy(
            routed_hbm.at[idx0.at[pl.ds(TPW + TSUB * c, TSUB)]], r1buf, semg1)
        if pend_y is not None:
            pend_y.wait()
        gs = pltpu.async_copy(shared_hbm.at[pl.ds(tbase, TSUB)], sbuf, semg2)
        g0.wait()
        g1.wait()
        gs.wait()

        def body(t, carry):
            for d in range(DIM // 16):
                sl = pl.ds(16 * d, 16)
                sbuf[t, sl] = sbuf[t, sl] + r0buf[t, sl] + r1buf[t, sl]
            return carry

        lax.fori_loop(0, TSUB, body, 0)
        pend_y = pltpu.async_copy(sbuf, yhbm.at[pl.ds(tbase, TSUB)], semy)
    pend_y.wait()


def _combine(shared, routed, dest):
    mesh = plsc.VectorSubcoreMesh(core_axis_name="c", subcore_axis_name="s")
    f = pl.kernel(
        _combine_body,
        out_type=jax.ShapeDtypeStruct((T, DIM), jnp.float32),
        mesh=mesh,
        scratch_types=[
            pltpu.VMEM((CHUNK,), jnp.int32),
            pltpu.VMEM((TSUB, DIM), jnp.float32),
            pltpu.VMEM((TSUB, DIM), jnp.float32),
            pltpu.VMEM((TSUB, DIM), jnp.float32),
            pltpu.SemaphoreType.DMA,
            pltpu.SemaphoreType.DMA,
            pltpu.SemaphoreType.DMA,
            pltpu.SemaphoreType.DMA,
        ],
    )
    return f(shared, routed, dest)


def kernel(x, gate_w, gate_b, W1, B1, W2, B2, W3, B3, SW1, SB1, SW2, SB2, SW3, SB3):
    bsz, seq, hdim = x.shape
    xf = x.reshape(-1, hdim)
    e2d, w116, w216 = _gate(xf, gate_w, gate_b)
    eperm = e2d.reshape(NW, TPW, 2).transpose(0, 2, 1).reshape(TS)
    wperm = jnp.stack([w116.reshape(NW, TPW, 128), w216.reshape(NW, TPW, 128)],
                      axis=1).reshape(TS, 128)
    dest, counts16 = _b1(eperm)
    sorted_x, wsorted = _b2(xf, wperm, dest)
    eid, blk, gs, gend, ini = _metadata(counts16)
    routed = _grouped(sorted_x, wsorted, W1, B1, W2, B2, W3, B3, eid, blk,
                      gs, gend, ini)
    shared = _shared(xf, SW1, SB1, SW2, SB2, SW3, SB3)
    y = _combine(shared, routed, dest)
    return y.reshape(bsz, seq, hdim)


# speedup vs baseline: 1.1056x; 1.0011x over previous
import functools

import jax
import jax.numpy as jnp
from jax import lax
from jax.experimental import pallas as pl
from jax.experimental.pallas import tpu as pltpu
from jax.experimental.pallas import tpu_sc as plsc

E = 8
DIM = 1024
INTER = 512
SINTER = 1024
T = 4096
TS = 2 * T
BLK = 256
NB = TS // BLK
NT = NB + E - 1
NW = 32
CHUNK = TS // NW
TPW = T // NW
SUB = 4
SUBN = CHUNK // SUB
TSUB = TPW // SUB


def _m2i(mask):
    return jnp.where(mask, jnp.ones((16,), jnp.int32),
                     jnp.zeros((16,), jnp.int32))


def _silu(v):
    return v * (1.0 / (1.0 + jnp.exp(-v)))


def _gate_block(x_ref, gw_ref, gb_ref, e_ref, w1_ref, w2_ref):
    xs = x_ref[...]
    logits = lax.dot_general(xs, gw_ref[...], (((1,), (1,)), ((), ())),
                             preferred_element_type=jnp.float32)
    logits = logits + gb_ref[...]
    m = jnp.max(logits, axis=1, keepdims=True)
    ex = jnp.exp(logits - m)
    scores = ex / jnp.sum(ex, axis=1, keepdims=True)
    cols = lax.broadcasted_iota(jnp.int32, scores.shape, 1)
    m1 = jnp.max(scores, axis=1, keepdims=True)
    i1 = jnp.min(jnp.where(scores == m1, cols, E), axis=1, keepdims=True)
    masked = jnp.where(cols == i1, -jnp.inf, scores)
    m2 = jnp.max(masked, axis=1, keepdims=True)
    i2 = jnp.min(jnp.where(masked == m2, cols, E), axis=1, keepdims=True)
    denom = m1 + m2 + 1e-20
    e_ref[...] = jnp.concatenate([i1, i2], axis=1)
    w1_ref[...] = jnp.broadcast_to(m1 / denom, w1_ref.shape)
    w2_ref[...] = jnp.broadcast_to(m2 / denom, w2_ref.shape)


def _gate(xf, gate_w, gate_b):
    bm = 1024
    return pl.pallas_call(
        _gate_block,
        grid=(T // bm,),
        in_specs=[
            pl.BlockSpec((bm, DIM), lambda i: (i, 0)),
            pl.BlockSpec((E, DIM), lambda i: (0, 0)),
            pl.BlockSpec((1, E), lambda i: (0, 0)),
        ],
        out_specs=[
            pl.BlockSpec((bm, 2), lambda i: (i, 0)),
            pl.BlockSpec((bm, 128), lambda i: (i, 0)),
            pl.BlockSpec((bm, 128), lambda i: (i, 0)),
        ],
        out_shape=[
            jax.ShapeDtypeStruct((T, 2), jnp.int32),
            jax.ShapeDtypeStruct((T, 128), jnp.float32),
            jax.ShapeDtypeStruct((T, 128), jnp.float32),
        ],
    )(xf, gate_w, gate_b.reshape(1, E))


def _b1_block(e_ref, dest_ref, cnt_ref):
    ev = e_ref[...]
    r128 = lax.broadcasted_iota(jnp.int32, (128, 128), 0)
    c128 = lax.broadcasted_iota(jnp.int32, (128, 128), 1)
    su = jnp.where(r128 < c128, 1.0, 0.0)
    r64 = lax.broadcasted_iota(jnp.int32, (64, 64), 0)
    c64 = lax.broadcasted_iota(jnp.int32, (64, 64), 1)
    sl = jnp.where(c64 < r64, 1.0, 0.0)
    dest = jnp.zeros(ev.shape, jnp.float32)
    cnts = jnp.zeros((1, 16), jnp.float32)
    lane16 = lax.broadcasted_iota(jnp.int32, (1, 16), 1)
    start = 0.0
    for e in range(E):
        m = jnp.where(ev == e, 1.0, 0.0)
        pw = lax.dot_general(m, su, (((1,), (0,)), ((), ())),
                             preferred_element_type=jnp.float32)
        s = jnp.sum(m, axis=1, keepdims=True)
        o = lax.dot_general(sl, s, (((1,), (0,)), ((), ())),
                            preferred_element_type=jnp.float32)
        rank = pw + o
        total = jnp.sum(s)
        dest = dest + m * (start + rank)
        cnts = cnts + jnp.where(lane16 == e, total, 0.0)
        start = start + total
    dest_ref[...] = dest.astype(jnp.int32)
    cnt_ref[...] = cnts.astype(jnp.int32)


def _b1(eperm):
    dest2d, cnt2d = pl.pallas_call(
        _b1_block,
        grid=(1,),
        in_specs=[pl.BlockSpec((64, 128), lambda i: (0, 0))],
        out_specs=[
            pl.BlockSpec((64, 128), lambda i: (0, 0)),
            pl.BlockSpec((1, 16), lambda i: (0, 0)),
        ],
        out_shape=[
            jax.ShapeDtypeStruct((64, 128), jnp.int32),
            jax.ShapeDtypeStruct((1, 16), jnp.int32),
        ],
    )(eperm.reshape(64, 128))
    return dest2d.reshape(TS), cnt2d.reshape(16)


B2SUB = 8
B2N = CHUNK // B2SUB


def _b2_body(xhbm, wperm, dest, sortedx, wsorted, didx0, didx1, idxv0, idxv1,
             rows0, rows1, wbuf0, wbuf1, semg0, semg1, sems0, sems1, semw0,
             semw1):
    wid = lax.axis_index("s") * 2 + lax.axis_index("c")
    base = wid * CHUNK
    lane = lax.broadcasted_iota(jnp.int32, (16,), 0)
    didx = [didx0, didx1]
    idxv = [idxv0, idxv1]
    rows = [rows0, rows1]
    wbuf = [wbuf0, wbuf1]
    semg = [semg0, semg1]
    sems = [sems0, sems1]
    semw = [semw0, semw1]
    pend_s = [None, None]
    pend_w = [None, None]

    for c in range(B2SUB):
        b = c & 1
        if pend_s[b] is not None:
            pend_s[b].wait()
            pend_w[b].wait()
        pltpu.sync_copy(dest.at[pl.ds(base + B2N * c, B2N)], didx[b])
        tb = wid * TPW + (B2N * c) % TPW
        for k in range(B2N // 16):
            idxv[b][pl.ds(16 * k, 16)] = lane + (tb + 16 * k)
        g = pltpu.async_copy(xhbm.at[idxv[b]], rows[b], semg[b])
        pltpu.sync_copy(wperm.at[pl.ds(base + B2N * c, B2N)], wbuf[b])
        g.wait()
        pend_s[b] = pltpu.async_copy(rows[b], sortedx.at[didx[b]], sems[b])
        pend_w[b] = pltpu.async_copy(wbuf[b], wsorted.at[didx[b]], semw[b])
    pend_s[0].wait()
    pend_w[0].wait()
    pend_s[1].wait()
    pend_w[1].wait()


def _b2(xf, wperm, dest):
    mesh = plsc.VectorSubcoreMesh(core_axis_name="c", subcore_axis_name="s")
    f = pl.kernel(
        _b2_body,
        out_type=[
            jax.ShapeDtypeStruct((TS, DIM), jnp.float32),
            jax.ShapeDtypeStruct((TS, 128), jnp.float32),
        ],
        mesh=mesh,
        scratch_types=[
            pltpu.VMEM((B2N,), jnp.int32),
            pltpu.VMEM((B2N,), jnp.int32),
            pltpu.VMEM((B2N,), jnp.int32),
            pltpu.VMEM((B2N,), jnp.int32),
            pltpu.VMEM((B2N, DIM), jnp.float32),
            pltpu.VMEM((B2N, DIM), jnp.float32),
            pltpu.VMEM((B2N, 128), jnp.float32),
            pltpu.VMEM((B2N, 128), jnp.float32),
            pltpu.SemaphoreType.DMA,
            pltpu.SemaphoreType.DMA,
            pltpu.SemaphoreType.DMA,
            pltpu.SemaphoreType.DMA,
            pltpu.SemaphoreType.DMA,
            pltpu.SemaphoreType.DMA,
        ],
    )
    return f(xf, wperm, dest)


def _metadata(counts16):
    cnt = counts16[:E]
    offs = jnp.concatenate(
        [jnp.zeros((1,), jnp.int32), jnp.cumsum(cnt, dtype=jnp.int32)])
    first_blk = offs[:E] // BLK
    last_blk = jnp.maximum(offs[1:] - 1, 0) // BLK
    nt_e = jnp.where(cnt > 0, last_blk - first_blk + 1, 0).astype(jnp.int32)
    tstart = jnp.concatenate(
        [jnp.zeros((1,), jnp.int32), jnp.cumsum(nt_e, dtype=jnp.int32)])
    ntact = tstart[E]
    tau = jnp.arange(NT, dtype=jnp.int32)
    eid = jnp.sum((tau[:, None] >= tstart[None, :E]).astype(jnp.int32),
                  axis=1) - 1
    eid = jnp.clip(eid, 0, E - 1)
    valid = tau < ntact
    blk = first_blk[eid] + (tau - tstart[eid])
    blk = jnp.where(valid, blk, NB - 1)
    gs = jnp.where(valid, offs[eid], TS)
    gend = jnp.where(valid, offs[eid + 1], TS)
    eid = jnp.where(valid, eid, E - 1)
    ini = jnp.concatenate([jnp.ones((1,), jnp.int32),
                           (blk[1:] != blk[:-1]).astype(jnp.int32)])
    ini = ini * valid.astype(jnp.int32)
    return eid, blk, gs, gend, ini


def _grouped_block(eid_r, blk_r, gs_r, ge_r, ini_r, xs_ref, ws_ref, w1_ref,
                   vb1_ref, w2_ref, vb2_ref, w3_ref, vb3_ref, out_ref):
    t = pl.program_id(0)
    xs = xs_ref[...]
    h1 = lax.dot_general(xs, w1_ref[0], (((1,), (1,)), ((), ())),
                         preferred_element_type=jnp.float32) + vb1_ref[0]
    h3 = lax.dot_general(xs, w3_ref[0], (((1,), (1,)), ((), ())),
                         preferred_element_type=jnp.float32) + vb3_ref[0]
    h = _silu(h1) * h3
    oe = lax.dot_general(h, w2_ref[0], (((1,), (1,)), ((), ())),
                         preferred_element_type=jnp.float32) + vb2_ref[0]
    oe = oe * ws_ref[:, 0:1]
    rows = blk_r[t] * BLK + lax.broadcasted_iota(jnp.int32, (BLK, 1), 0)
    active = (rows >= gs_r[t]) & (rows < ge_r[t])
    oe = jnp.where(active, oe, 0.0)

    @pl.when(ini_r[t] == 1)
    def _():
        out_ref[...] = oe

    @pl.when(ini_r[t] == 0)
    def _():
        out_ref[...] = out_ref[...] + oe


def _grouped(sorted_x, wsorted, W1, B1, W2, B2, W3, B3, eid, blk, gs, gend,
             ini):
    grid_spec = pltpu.PrefetchScalarGridSpec(
        num_scalar_prefetch=5,
        grid=(NT,),
        in_specs=[
            pl.BlockSpec((BLK, DIM), lambda t, ei, bl, g0, g1, i0: (bl[t], 0)),
            pl.BlockSpec((BLK, 128), lambda t, ei, bl, g0, g1, i0: (bl[t], 0)),
            pl.BlockSpec((1, INTER, DIM),
                         lambda t, ei, bl, g0, g1, i0: (ei[t], 0, 0)),
            pl.BlockSpec((1, 1, INTER),
                         lambda t, ei, bl, g0, g1, i0: (ei[t], 0, 0)),
            pl.BlockSpec((1, DIM, INTER),
                         lambda t, ei, bl, g0, g1, i0: (ei[t], 0, 0)),
            pl.BlockSpec((1, 1, DIM),
                         lambda t, ei, bl, g0, g1, i0: (ei[t], 0, 0)),
            pl.BlockSpec((1, INTER, DIM),
                         lambda t, ei, bl, g0, g1, i0: (ei[t], 0, 0)),
            pl.BlockSpec((1, 1, INTER),
                         lambda t, ei, bl, g0, g1, i0: (ei[t], 0, 0)),
        ],
        out_specs=pl.BlockSpec((BLK, DIM),
                               lambda t, ei, bl, g0, g1, i0: (bl[t], 0)),
    )
    return pl.pallas_call(
        _grouped_block,
        grid_spec=grid_spec,
        out_shape=jax.ShapeDtypeStruct((TS, DIM), jnp.float32),
        compiler_params=pltpu.CompilerParams(
            dimension_semantics=("arbitrary",)),
    )(eid, blk, gs, gend, ini, sorted_x, wsorted, W1, B1[:, None], W2,
      B2[:, None], W3, B3[:, None])


def _shared_block(x_ref, w1_ref, b1_ref, w2_ref, b2_ref, w3_ref, b3_ref,
                  out_ref):
    xs = x_ref[...]
    h1 = lax.dot_general(xs, w1_ref[...], (((1,), (1,)), ((), ())),
                         preferred_element_type=jnp.float32) + b1_ref[...]
    h3 = lax.dot_general(xs, w3_ref[...], (((1,), (1,)), ((), ())),
                         preferred_element_type=jnp.float32) + b3_ref[...]
    h = _silu(h1) * h3
    out_ref[...] = lax.dot_general(h, w2_ref[...], (((1,), (1,)), ((), ())),
                                   preferred_element_type=jnp.float32
                                   ) + b2_ref[...]


def _shared(xf, SW1, SB1, SW2, SB2, SW3, SB3):
    bm = 1024
    return pl.pallas_call(
        _shared_block,
        grid=(T // bm,),
        in_specs=[
            pl.BlockSpec((bm, DIM), lambda i: (i, 0)),
            pl.BlockSpec((SINTER, DIM), lambda i: (0, 0)),
            pl.BlockSpec((1, SINTER), lambda i: (0, 0)),
            pl.BlockSpec((DIM, SINTER), lambda i: (0, 0)),
            pl.BlockSpec((1, DIM), lambda i: (0, 0)),
            pl.BlockSpec((SINTER, DIM), lambda i: (0, 0)),
            pl.BlockSpec((1, SINTER), lambda i: (0, 0)),
        ],
        out_specs=pl.BlockSpec((bm, DIM), lambda i: (i, 0)),
        out_shape=jax.ShapeDtypeStruct((T, DIM), jnp.float32),
    )(xf, SW1, SB1.reshape(1, SINTER), SW2, SB2.reshape(1, DIM),
      SW3, SB3.reshape(1, SINTER))


def _combine_body(shared_hbm, routed_hbm, dest, yhbm, idx0, r0buf,
                  r1buf, sbuf, semg0, semg1, semg2, semy):
    wid = lax.axis_index("s") * 2 + lax.axis_index("c")
    base = wid * CHUNK

    pltpu.sync_copy(dest.at[pl.ds(base, CHUNK)], idx0)
    pend_y = None
    for c in range(SUB):
        tbase = wid * TPW + TSUB * c
        g0 = pltpu.async_copy(
            routed_hbm.at[idx0.at[pl.ds(TSUB * c, TSUB)]], r0buf, semg0)
        g1 = pltpu.async_copy(
            routed_hbm.at[idx0.at[pl.ds(TPW + TSUB * c, TSUB)]], r1buf, semg1)
        if pend_y is not None:
            pend_y.wait()
        gs = pltpu.async_copy(shared_hbm.at[pl.ds(tbase, TSUB)], sbuf, semg2)
        g0.wait()
        g1.wait()
        gs.wait()

        def body(t, carry):
            for d in range(DIM // 16):
                sl = pl.ds(16 * d, 16)
                sbuf[t, sl] = sbuf[t, sl] + r0buf[t, sl] + r1buf[t, sl]
            return carry

        lax.fori_loop(0, TSUB, body, 0)
        pend_y = pltpu.async_copy(sbuf, yhbm.at[pl.ds(tbase, TSUB)], semy)
    pend_y.wait()


def _combine(shared, routed, dest):
    mesh = plsc.VectorSubcoreMesh(core_axis_name="c", subcore_axis_name="s")
    f = pl.kernel(
        _combine_body,
        out_type=jax.ShapeDtypeStruct((T, DIM), jnp.float32),
        mesh=mesh,
        scratch_types=[
            pltpu.VMEM((CHUNK,), jnp.int32),
            pltpu.VMEM((TSUB, DIM), jnp.float32),
            pltpu.VMEM((TSUB, DIM), jnp.float32),
            pltpu.VMEM((TSUB, DIM), jnp.float32),
            pltpu.SemaphoreType.DMA,
            pltpu.SemaphoreType.DMA,
            pltpu.SemaphoreType.DMA,
            pltpu.SemaphoreType.DMA,
        ],
    )
    return f(shared, routed, dest)


def kernel(x, gate_w, gate_b, W1, B1, W2, B2, W3, B3, SW1, SB1, SW2, SB2, SW3, SB3):
    bsz, seq, hdim = x.shape
    xf = x.reshape(-1, hdim)
    e2d, w116, w216 = _gate(xf, gate_w, gate_b)
    eperm = e2d.reshape(NW, TPW, 2).transpose(0, 2, 1).reshape(TS)
    wperm = jnp.stack([w116.reshape(NW, TPW, 128), w216.reshape(NW, TPW, 128)],
                      axis=1).reshape(TS, 128)
    dest, counts16 = _b1(eperm)
    sorted_x, wsorted = _b2(xf, wperm, dest)
    shared = _shared(xf, SW1, SB1, SW2, SB2, SW3, SB3)
    eid, blk, gs, gend, ini = _metadata(counts16)
    routed = _grouped(sorted_x, wsorted, W1, B1, W2, B2, W3, B3, eid, blk,
                      gs, gend, ini)
    y = _combine(shared, routed, dest)
    return y.reshape(bsz, seq, hdim)


# speedup vs baseline: 1.1850x; 1.0718x over previous
import functools

import jax
import jax.numpy as jnp
from jax import lax
from jax.experimental import pallas as pl
from jax.experimental.pallas import tpu as pltpu
from jax.experimental.pallas import tpu_sc as plsc

E = 8
DIM = 1024
INTER = 512
SINTER = 1024
T = 4096
TS = 2 * T
BLK = 256
NB = TS // BLK
NT = NB + E - 1
NW = 32
CHUNK = TS // NW
TPW = T // NW
SUB = 4
SUBN = CHUNK // SUB
TSUB = TPW // SUB


def _m2i(mask):
    return jnp.where(mask, jnp.ones((16,), jnp.int32),
                     jnp.zeros((16,), jnp.int32))


def _silu(v):
    return v * (1.0 / (1.0 + jnp.exp(-v)))


def _gate_block(x_ref, gw_ref, gb_ref, e_ref, w1_ref, w2_ref):
    xs = x_ref[...]
    logits = lax.dot_general(xs, gw_ref[...], (((1,), (1,)), ((), ())),
                             preferred_element_type=jnp.float32)
    logits = logits + gb_ref[...]
    m = jnp.max(logits, axis=1, keepdims=True)
    ex = jnp.exp(logits - m)
    scores = ex / jnp.sum(ex, axis=1, keepdims=True)
    cols = lax.broadcasted_iota(jnp.int32, scores.shape, 1)
    m1 = jnp.max(scores, axis=1, keepdims=True)
    i1 = jnp.min(jnp.where(scores == m1, cols, E), axis=1, keepdims=True)
    masked = jnp.where(cols == i1, -jnp.inf, scores)
    m2 = jnp.max(masked, axis=1, keepdims=True)
    i2 = jnp.min(jnp.where(masked == m2, cols, E), axis=1, keepdims=True)
    denom = m1 + m2 + 1e-20
    e_ref[...] = jnp.concatenate([i1, i2], axis=1)
    w1_ref[...] = jnp.broadcast_to(m1 / denom, w1_ref.shape)
    w2_ref[...] = jnp.broadcast_to(m2 / denom, w2_ref.shape)


def _gate(xf, gate_w, gate_b):
    bm = 1024
    return pl.pallas_call(
        _gate_block,
        grid=(T // bm,),
        in_specs=[
            pl.BlockSpec((bm, DIM), lambda i: (i, 0)),
            pl.BlockSpec((E, DIM), lambda i: (0, 0)),
            pl.BlockSpec((1, E), lambda i: (0, 0)),
        ],
        out_specs=[
            pl.BlockSpec((bm, 2), lambda i: (i, 0)),
            pl.BlockSpec((bm, 128), lambda i: (i, 0)),
            pl.BlockSpec((bm, 128), lambda i: (i, 0)),
        ],
        out_shape=[
            jax.ShapeDtypeStruct((T, 2), jnp.int32),
            jax.ShapeDtypeStruct((T, 128), jnp.float32),
            jax.ShapeDtypeStruct((T, 128), jnp.float32),
        ],
    )(xf, gate_w, gate_b.reshape(1, E))


def _b1_block(e_ref, dest_ref, cnt_ref):
    ev = e_ref[...]
    r128 = lax.broadcasted_iota(jnp.int32, (128, 128), 0)
    c128 = lax.broadcasted_iota(jnp.int32, (128, 128), 1)
    su = jnp.where(r128 < c128, 1.0, 0.0)
    r64 = lax.broadcasted_iota(jnp.int32, (64, 64), 0)
    c64 = lax.broadcasted_iota(jnp.int32, (64, 64), 1)
    sl = jnp.where(c64 < r64, 1.0, 0.0)
    dest = jnp.zeros(ev.shape, jnp.float32)
    cnts = jnp.zeros((1, 16), jnp.float32)
    lane16 = lax.broadcasted_iota(jnp.int32, (1, 16), 1)
    start = 0.0
    for e in range(E):
        m = jnp.where(ev == e, 1.0, 0.0)
        pw = lax.dot_general(m, su, (((1,), (0,)), ((), ())),
                             preferred_element_type=jnp.float32)
        s = jnp.sum(m, axis=1, keepdims=True)
        o = lax.dot_general(sl, s, (((1,), (0,)), ((), ())),
                            preferred_element_type=jnp.float32)
        rank = pw + o
        total = jnp.sum(s)
        dest = dest + m * (start + rank)
        cnts = cnts + jnp.where(lane16 == e, total, 0.0)
        start = start + total
    dest_ref[...] = dest.astype(jnp.int32)
    cnt_ref[...] = cnts.astype(jnp.int32)


def _b1(eperm):
    dest2d, cnt2d = pl.pallas_call(
        _b1_block,
        grid=(1,),
        in_specs=[pl.BlockSpec((64, 128), lambda i: (0, 0))],
        out_specs=[
            pl.BlockSpec((64, 128), lambda i: (0, 0)),
            pl.BlockSpec((1, 16), lambda i: (0, 0)),
        ],
        out_shape=[
            jax.ShapeDtypeStruct((64, 128), jnp.int32),
            jax.ShapeDtypeStruct((1, 16), jnp.int32),
        ],
    )(eperm.reshape(64, 128))
    return dest2d.reshape(TS), cnt2d.reshape(16)


B2SUB = 8
B2N = CHUNK // B2SUB


def _b2_body(xhbm, wperm, dest, sortedx, wsorted, didx0, didx1, idxv0, idxv1,
             rows0, rows1, wbuf0, wbuf1, semg0, semg1, sems0, sems1, semw0,
             semw1):
    wid = lax.axis_index("s") * 2 + lax.axis_index("c")
    base = wid * CHUNK
    lane = lax.broadcasted_iota(jnp.int32, (16,), 0)
    didx = [didx0, didx1]
    idxv = [idxv0, idxv1]
    rows = [rows0, rows1]
    wbuf = [wbuf0, wbuf1]
    semg = [semg0, semg1]
    sems = [sems0, sems1]
    semw = [semw0, semw1]
    pend_s = [None, None]
    pend_w = [None, None]

    for c in range(B2SUB):
        b = c & 1
        if pend_s[b] is not None:
            pend_s[b].wait()
            pend_w[b].wait()
        pltpu.sync_copy(dest.at[pl.ds(base + B2N * c, B2N)], didx[b])
        tb = wid * TPW + (B2N * c) % TPW
        for k in range(B2N // 16):
            idxv[b][pl.ds(16 * k, 16)] = lane + (tb + 16 * k)
        g = pltpu.async_copy(xhbm.at[idxv[b]], rows[b], semg[b])
        pltpu.sync_copy(wperm.at[pl.ds(base + B2N * c, B2N)], wbuf[b])
        g.wait()
        pend_s[b] = pltpu.async_copy(rows[b], sortedx.at[didx[b]], sems[b])
        pend_w[b] = pltpu.async_copy(wbuf[b], wsorted.at[didx[b]], semw[b])
    pend_s[0].wait()
    pend_w[0].wait()
    pend_s[1].wait()
    pend_w[1].wait()


def _b2(xf, wperm, dest):
    mesh = plsc.VectorSubcoreMesh(core_axis_name="c", subcore_axis_name="s")
    f = pl.kernel(
        _b2_body,
        out_type=[
            jax.ShapeDtypeStruct((TS, DIM), jnp.float32),
            jax.ShapeDtypeStruct((TS, 128), jnp.float32),
        ],
        mesh=mesh,
        scratch_types=[
            pltpu.VMEM((B2N,), jnp.int32),
            pltpu.VMEM((B2N,), jnp.int32),
            pltpu.VMEM((B2N,), jnp.int32),
            pltpu.VMEM((B2N,), jnp.int32),
            pltpu.VMEM((B2N, DIM), jnp.float32),
            pltpu.VMEM((B2N, DIM), jnp.float32),
            pltpu.VMEM((B2N, 128), jnp.float32),
            pltpu.VMEM((B2N, 128), jnp.float32),
            pltpu.SemaphoreType.DMA,
            pltpu.SemaphoreType.DMA,
            pltpu.SemaphoreType.DMA,
            pltpu.SemaphoreType.DMA,
            pltpu.SemaphoreType.DMA,
            pltpu.SemaphoreType.DMA,
        ],
    )
    return f(xf, wperm, dest)


def _metadata(counts16):
    cnt = counts16[:E]
    offs = jnp.concatenate(
        [jnp.zeros((1,), jnp.int32), jnp.cumsum(cnt, dtype=jnp.int32)])
    first_blk = offs[:E] // BLK
    last_blk = jnp.maximum(offs[1:] - 1, 0) // BLK
    nt_e = jnp.where(cnt > 0, last_blk - first_blk + 1, 0).astype(jnp.int32)
    tstart = jnp.concatenate(
        [jnp.zeros((1,), jnp.int32), jnp.cumsum(nt_e, dtype=jnp.int32)])
    ntact = tstart[E]
    tau = jnp.arange(NT, dtype=jnp.int32)
    eid = jnp.sum((tau[:, None] >= tstart[None, :E]).astype(jnp.int32),
                  axis=1) - 1
    eid = jnp.clip(eid, 0, E - 1)
    valid = tau < ntact
    blk = first_blk[eid] + (tau - tstart[eid])
    blk = jnp.where(valid, blk, NB - 1)
    gs = jnp.where(valid, offs[eid], TS)
    gend = jnp.where(valid, offs[eid + 1], TS)
    eid = jnp.where(valid, eid, E - 1)
    ini = jnp.concatenate([jnp.ones((1,), jnp.int32),
                           (blk[1:] != blk[:-1]).astype(jnp.int32)])
    ini = ini * valid.astype(jnp.int32)
    return eid, blk, gs, gend, ini


def _grouped_block(eid_r, blk_r, gs_r, ge_r, ini_r, xs_ref, ws_ref, w1_ref,
                   vb1_ref, w2_ref, vb2_ref, w3_ref, vb3_ref, out_ref):
    t = pl.program_id(0)
    xs = xs_ref[...]
    h1 = lax.dot_general(xs, w1_ref[0], (((1,), (1,)), ((), ())),
                         preferred_element_type=jnp.float32) + vb1_ref[0]
    h3 = lax.dot_general(xs, w3_ref[0], (((1,), (1,)), ((), ())),
                         preferred_element_type=jnp.float32) + vb3_ref[0]
    h = _silu(h1) * h3
    oe = lax.dot_general(h, w2_ref[0], (((1,), (1,)), ((), ())),
                         preferred_element_type=jnp.float32) + vb2_ref[0]
    oe = oe * ws_ref[:, 0:1]
    rows = blk_r[t] * BLK + lax.broadcasted_iota(jnp.int32, (BLK, 1), 0)
    active = (rows >= gs_r[t]) & (rows < ge_r[t])
    oe = jnp.where(active, oe, 0.0)

    @pl.when(ini_r[t] == 1)
    def _():
        out_ref[...] = oe

    @pl.when(ini_r[t] == 0)
    def _():
        out_ref[...] = out_ref[...] + oe


def _grouped(sorted_x, wsorted, W1, B1, W2, B2, W3, B3, eid, blk, gs, gend,
             ini):
    grid_spec = pltpu.PrefetchScalarGridSpec(
        num_scalar_prefetch=5,
        grid=(NT,),
        in_specs=[
            pl.BlockSpec((BLK, DIM), lambda t, ei, bl, g0, g1, i0: (bl[t], 0)),
            pl.BlockSpec((BLK, 128), lambda t, ei, bl, g0, g1, i0: (bl[t], 0)),
            pl.BlockSpec((1, INTER, DIM),
                         lambda t, ei, bl, g0, g1, i0: (ei[t], 0, 0)),
            pl.BlockSpec((1, 1, INTER),
                         lambda t, ei, bl, g0, g1, i0: (ei[t], 0, 0)),
            pl.BlockSpec((1, DIM, INTER),
                         lambda t, ei, bl, g0, g1, i0: (ei[t], 0, 0)),
            pl.BlockSpec((1, 1, DIM),
                         lambda t, ei, bl, g0, g1, i0: (ei[t], 0, 0)),
            pl.BlockSpec((1, INTER, DIM),
                         lambda t, ei, bl, g0, g1, i0: (ei[t], 0, 0)),
            pl.BlockSpec((1, 1, INTER),
                         lambda t, ei, bl, g0, g1, i0: (ei[t], 0, 0)),
        ],
        out_specs=pl.BlockSpec((BLK, DIM),
                               lambda t, ei, bl, g0, g1, i0: (bl[t], 0)),
    )
    return pl.pallas_call(
        _grouped_block,
        grid_spec=grid_spec,
        out_shape=jax.ShapeDtypeStruct((TS, DIM), jnp.float32),
        compiler_params=pltpu.CompilerParams(
            dimension_semantics=("arbitrary",)),
    )(eid, blk, gs, gend, ini, sorted_x, wsorted, W1, B1[:, None], W2,
      B2[:, None], W3, B3[:, None])


def _shared_block(x_ref, w1_ref, b1_ref, w2_ref, b2_ref, w3_ref, b3_ref,
                  out_ref):
    xs = x_ref[...]
    h1 = lax.dot_general(xs, w1_ref[...], (((1,), (1,)), ((), ())),
                         preferred_element_type=jnp.float32) + b1_ref[...]
    h3 = lax.dot_general(xs, w3_ref[...], (((1,), (1,)), ((), ())),
                         preferred_element_type=jnp.float32) + b3_ref[...]
    h = _silu(h1) * h3
    out_ref[...] = lax.dot_general(h, w2_ref[...], (((1,), (1,)), ((), ())),
                                   preferred_element_type=jnp.float32
                                   ) + b2_ref[...]


def _shared(xf, SW1, SB1, SW2, SB2, SW3, SB3):
    bm = 1024
    return pl.pallas_call(
        _shared_block,
        grid=(T // bm,),
        in_specs=[
            pl.BlockSpec((bm, DIM), lambda i: (i, 0)),
            pl.BlockSpec((SINTER, DIM), lambda i: (0, 0)),
            pl.BlockSpec((1, SINTER), lambda i: (0, 0)),
            pl.BlockSpec((DIM, SINTER), lambda i: (0, 0)),
            pl.BlockSpec((1, DIM), lambda i: (0, 0)),
            pl.BlockSpec((SINTER, DIM), lambda i: (0, 0)),
            pl.BlockSpec((1, SINTER), lambda i: (0, 0)),
        ],
        out_specs=pl.BlockSpec((bm, DIM), lambda i: (i, 0)),
        out_shape=jax.ShapeDtypeStruct((T, DIM), jnp.float32),
    )(xf, SW1, SB1.reshape(1, SINTER), SW2, SB2.reshape(1, DIM),
      SW3, SB3.reshape(1, SINTER))


def _combine_body(shared_hbm, routed_hbm, dest, yhbm, idx0,
                  r0a, r0b, r1a, r1b, sa, sb,
                  sg0a, sg0b, sg1a, sg1b, sg2a, sg2b, sya, syb):
    wid = lax.axis_index("s") * 2 + lax.axis_index("c")
    base = wid * CHUNK
    EN = 16
    ESUB = TPW // EN
    r0 = [r0a, r0b]
    r1 = [r1a, r1b]
    sB = [sa, sb]
    sg0 = [sg0a, sg0b]
    sg1 = [sg1a, sg1b]
    sg2 = [sg2a, sg2b]
    sy = [sya, syb]

    pltpu.sync_copy(dest.at[pl.ds(base, CHUNK)], idx0)

    def issue(c):
        b = c & 1
        tbase = wid * TPW + EN * c
        g0 = pltpu.async_copy(
            routed_hbm.at[idx0.at[pl.ds(EN * c, EN)]], r0[b], sg0[b])
        g1 = pltpu.async_copy(
            routed_hbm.at[idx0.at[pl.ds(TPW + EN * c, EN)]], r1[b], sg1[b])
        g2 = pltpu.async_copy(shared_hbm.at[pl.ds(tbase, EN)], sB[b], sg2[b])
        return (g0, g1, g2)

    pend_y = [None, None]
    h = issue(0)
    for c in range(ESUB):
        b = c & 1
        if c + 1 < ESUB:
            nb = (c + 1) & 1
            if pend_y[nb] is not None:
                pend_y[nb].wait()
            h_next = issue(c + 1)
        h[0].wait()
        h[1].wait()
        h[2].wait()

        def body(t, carry):
            for d in range(DIM // 16):
                sl = pl.ds(16 * d, 16)
                sB[b][t, sl] = sB[b][t, sl] + r0[b][t, sl] + r1[b][t, sl]
            return carry

        lax.fori_loop(0, EN, body, 0)
        tbase = wid * TPW + EN * c
        pend_y[b] = pltpu.async_copy(sB[b], yhbm.at[pl.ds(tbase, EN)], sy[b])
        if c + 1 < ESUB:
            h = h_next
    pend_y[0].wait()
    pend_y[1].wait()


def _combine(shared, routed, dest):
    mesh = plsc.VectorSubcoreMesh(core_axis_name="c", subcore_axis_name="s")
    f = pl.kernel(
        _combine_body,
        out_type=jax.ShapeDtypeStruct((T, DIM), jnp.float32),
        mesh=mesh,
        scratch_types=(
            [pltpu.VMEM((CHUNK,), jnp.int32)]
            + [pltpu.VMEM((16, DIM), jnp.float32)] * 6
            + [pltpu.SemaphoreType.DMA] * 8
        ),
    )
    return f(shared, routed, dest)


def kernel(x, gate_w, gate_b, W1, B1, W2, B2, W3, B3, SW1, SB1, SW2, SB2, SW3, SB3):
    bsz, seq, hdim = x.shape
    xf = x.reshape(-1, hdim)
    e2d, w116, w216 = _gate(xf, gate_w, gate_b)
    eperm = e2d.reshape(NW, TPW, 2).transpose(0, 2, 1).reshape(TS)
    wperm = jnp.stack([w116.reshape(NW, TPW, 128), w216.reshape(NW, TPW, 128)],
                      axis=1).reshape(TS, 128)
    dest, counts16 = _b1(eperm)
    sorted_x, wsorted = _b2(xf, wperm, dest)
    shared = _shared(xf, SW1, SB1, SW2, SB2, SW3, SB3)
    eid, blk, gs, gend, ini = _metadata(counts16)
    routed = _grouped(sorted_x, wsorted, W1, B1, W2, B2, W3, B3, eid, blk,
                      gs, gend, ini)
    y = _combine(shared, routed, dest)
    return y.reshape(bsz, seq, hdim)


# speedup vs baseline: 1.1886x; 1.0031x over previous
import jax
import jax.numpy as jnp
from jax import lax
from jax.experimental import pallas as pl
from jax.experimental.pallas import tpu as pltpu
from jax.experimental.pallas import tpu_sc as plsc

E = 8
DIM = 1024
INTER = 512
SINTER = 1024
T = 4096
TS = 2 * T
BLK = 256
NB = TS // BLK
NT = NB + E - 1
NW = 32
CHUNK = TS // NW
TPW = T // NW
SUB = 4
SUBN = CHUNK // SUB
TSUB = TPW // SUB


def _m2i(mask):
    return jnp.where(mask, jnp.ones((16,), jnp.int32),
                     jnp.zeros((16,), jnp.int32))


def _silu(v):
    return v * (1.0 / (1.0 + jnp.exp(-v)))


def _gate_block(x_ref, gw_ref, gb_ref, e_ref, w1_ref, w2_ref):
    xs = x_ref[...]
    logits = lax.dot_general(xs, gw_ref[...], (((1,), (1,)), ((), ())),
                             preferred_element_type=jnp.float32)
    logits = logits + gb_ref[...]
    m = jnp.max(logits, axis=1, keepdims=True)
    ex = jnp.exp(logits - m)
    scores = ex / jnp.sum(ex, axis=1, keepdims=True)
    cols = lax.broadcasted_iota(jnp.int32, scores.shape, 1)
    m1 = jnp.max(scores, axis=1, keepdims=True)
    i1 = jnp.min(jnp.where(scores == m1, cols, E), axis=1, keepdims=True)
    masked = jnp.where(cols == i1, -jnp.inf, scores)
    m2 = jnp.max(masked, axis=1, keepdims=True)
    i2 = jnp.min(jnp.where(masked == m2, cols, E), axis=1, keepdims=True)
    denom = m1 + m2 + 1e-20
    e_ref[...] = jnp.concatenate([i1, i2], axis=1)
    w1_ref[...] = jnp.broadcast_to(m1 / denom, w1_ref.shape)
    w2_ref[...] = jnp.broadcast_to(m2 / denom, w2_ref.shape)


def _gate(xf, gate_w, gate_b):
    bm = 1024
    return pl.pallas_call(
        _gate_block,
        grid=(T // bm,),
        in_specs=[
            pl.BlockSpec((bm, DIM), lambda i: (i, 0)),
            pl.BlockSpec((E, DIM), lambda i: (0, 0)),
            pl.BlockSpec((1, E), lambda i: (0, 0)),
        ],
        out_specs=[
            pl.BlockSpec((bm, 2), lambda i: (i, 0)),
            pl.BlockSpec((bm, 128), lambda i: (i, 0)),
            pl.BlockSpec((bm, 128), lambda i: (i, 0)),
        ],
        out_shape=[
            jax.ShapeDtypeStruct((T, 2), jnp.int32),
            jax.ShapeDtypeStruct((T, 128), jnp.float32),
            jax.ShapeDtypeStruct((T, 128), jnp.float32),
        ],
    )(xf, gate_w, gate_b.reshape(1, E))


def _b1_block(e_ref, dest_ref, cnt_ref):
    ev = e_ref[...]
    r128 = lax.broadcasted_iota(jnp.int32, (128, 128), 0)
    c128 = lax.broadcasted_iota(jnp.int32, (128, 128), 1)
    su = jnp.where(r128 < c128, 1.0, 0.0)
    r64 = lax.broadcasted_iota(jnp.int32, (64, 64), 0)
    c64 = lax.broadcasted_iota(jnp.int32, (64, 64), 1)
    sl = jnp.where(c64 < r64, 1.0, 0.0)
    dest = jnp.zeros(ev.shape, jnp.float32)
    cnts = jnp.zeros((1, 16), jnp.float32)
    lane16 = lax.broadcasted_iota(jnp.int32, (1, 16), 1)
    start = 0.0
    for e in range(E):
        m = jnp.where(ev == e, 1.0, 0.0)
        pw = lax.dot_general(m, su, (((1,), (0,)), ((), ())),
                             preferred_element_type=jnp.float32)
        s = jnp.sum(m, axis=1, keepdims=True)
        o = lax.dot_general(sl, s, (((1,), (0,)), ((), ())),
                            preferred_element_type=jnp.float32)
        rank = pw + o
        total = jnp.sum(s)
        dest = dest + m * (start + rank)
        cnts = cnts + jnp.where(lane16 == e, total, 0.0)
        start = start + total
    dest_ref[...] = dest.astype(jnp.int32)
    cnt_ref[...] = cnts.astype(jnp.int32)


def _b1(eperm):
    dest2d, cnt2d = pl.pallas_call(
        _b1_block,
        grid=(1,),
        in_specs=[pl.BlockSpec((64, 128), lambda i: (0, 0))],
        out_specs=[
            pl.BlockSpec((64, 128), lambda i: (0, 0)),
            pl.BlockSpec((1, 16), lambda i: (0, 0)),
        ],
        out_shape=[
            jax.ShapeDtypeStruct((64, 128), jnp.int32),
            jax.ShapeDtypeStruct((1, 16), jnp.int32),
        ],
    )(eperm.reshape(64, 128))
    return dest2d.reshape(TS), cnt2d.reshape(16)


B2SUB = 8
B2N = CHUNK // B2SUB


def _b2_body(xhbm, wperm, dest, sortedx, wsorted, didx0, didx1, idxv0, idxv1,
             rows0, rows1, wbuf0, wbuf1, semg0, semg1, sems0, sems1, semw0,
             semw1):
    wid = lax.axis_index("s") * 2 + lax.axis_index("c")
    base = wid * CHUNK
    lane = lax.broadcasted_iota(jnp.int32, (16,), 0)
    didx = [didx0, didx1]
    idxv = [idxv0, idxv1]
    rows = [rows0, rows1]
    wbuf = [wbuf0, wbuf1]
    semg = [semg0, semg1]
    sems = [sems0, sems1]
    semw = [semw0, semw1]
    pend_s = [None, None]
    pend_w = [None, None]

    for c in range(B2SUB):
        b = c & 1
        if pend_s[b] is not None:
            pend_s[b].wait()
            pend_w[b].wait()
        pltpu.sync_copy(dest.at[pl.ds(base + B2N * c, B2N)], didx[b])
        tb = wid * TPW + (B2N * c) % TPW
        for k in range(B2N // 16):
            idxv[b][pl.ds(16 * k, 16)] = lane + (tb + 16 * k)
        g = pltpu.async_copy(xhbm.at[idxv[b]], rows[b], semg[b])
        pltpu.sync_copy(wperm.at[pl.ds(base + B2N * c, B2N)], wbuf[b])
        g.wait()
        pend_s[b] = pltpu.async_copy(rows[b], sortedx.at[didx[b]], sems[b])
        pend_w[b] = pltpu.async_copy(wbuf[b], wsorted.at[didx[b]], semw[b])
    pend_s[0].wait()
    pend_w[0].wait()
    pend_s[1].wait()
    pend_w[1].wait()


def _b2(xf, wperm, dest):
    mesh = plsc.VectorSubcoreMesh(core_axis_name="c", subcore_axis_name="s")
    f = pl.kernel(
        _b2_body,
        out_type=[
            jax.ShapeDtypeStruct((TS, DIM), jnp.float32),
            jax.ShapeDtypeStruct((TS, 128), jnp.float32),
        ],
        mesh=mesh,
        scratch_types=[
            pltpu.VMEM((B2N,), jnp.int32),
            pltpu.VMEM((B2N,), jnp.int32),
            pltpu.VMEM((B2N,), jnp.int32),
            pltpu.VMEM((B2N,), jnp.int32),
            pltpu.VMEM((B2N, DIM), jnp.float32),
            pltpu.VMEM((B2N, DIM), jnp.float32),
            pltpu.VMEM((B2N, 128), jnp.float32),
            pltpu.VMEM((B2N, 128), jnp.float32),
            pltpu.SemaphoreType.DMA,
            pltpu.SemaphoreType.DMA,
            pltpu.SemaphoreType.DMA,
            pltpu.SemaphoreType.DMA,
            pltpu.SemaphoreType.DMA,
            pltpu.SemaphoreType.DMA,
        ],
    )
    return f(xf, wperm, dest)


def _metadata(counts16):
    cnt = counts16[:E]
    offs = jnp.concatenate(
        [jnp.zeros((1,), jnp.int32), jnp.cumsum(cnt, dtype=jnp.int32)])
    first_blk = offs[:E] // BLK
    last_blk = jnp.maximum(offs[1:] - 1, 0) // BLK
    nt_e = jnp.where(cnt > 0, last_blk - first_blk + 1, 0).astype(jnp.int32)
    tstart = jnp.concatenate(
        [jnp.zeros((1,), jnp.int32), jnp.cumsum(nt_e, dtype=jnp.int32)])
    ntact = tstart[E]
    tau = jnp.arange(NT, dtype=jnp.int32)
    eid = jnp.sum((tau[:, None] >= tstart[None, :E]).astype(jnp.int32),
                  axis=1) - 1
    eid = jnp.clip(eid, 0, E - 1)
    valid = tau < ntact
    blk = first_blk[eid] + (tau - tstart[eid])
    blk = jnp.where(valid, blk, NB - 1)
    gs = jnp.where(valid, offs[eid], TS)
    gend = jnp.where(valid, offs[eid + 1], TS)
    eid = jnp.where(valid, eid, E - 1)
    ini = jnp.concatenate([jnp.ones((1,), jnp.int32),
                           (blk[1:] != blk[:-1]).astype(jnp.int32)])
    ini = ini * valid.astype(jnp.int32)
    return eid, blk, gs, gend, ini


def _grouped_block(eid_r, blk_r, gs_r, ge_r, ini_r, xs_ref, ws_ref, w1_ref,
                   vb1_ref, w2_ref, vb2_ref, w3_ref, vb3_ref, out_ref):
    t = pl.program_id(0)
    xs = xs_ref[...]
    h1 = lax.dot_general(xs, w1_ref[0], (((1,), (1,)), ((), ())),
                         preferred_element_type=jnp.float32) + vb1_ref[0]
    h3 = lax.dot_general(xs, w3_ref[0], (((1,), (1,)), ((), ())),
                         preferred_element_type=jnp.float32) + vb3_ref[0]
    h = _silu(h1) * h3
    oe = lax.dot_general(h, w2_ref[0], (((1,), (1,)), ((), ())),
                         preferred_element_type=jnp.float32) + vb2_ref[0]
    oe = oe * ws_ref[:, 0:1]
    rows = blk_r[t] * BLK + lax.broadcasted_iota(jnp.int32, (BLK, 1), 0)
    active = (rows >= gs_r[t]) & (rows < ge_r[t])
    oe = jnp.where(active, oe, 0.0)

    @pl.when(ini_r[t] == 1)
    def _():
        out_ref[...] = oe

    @pl.when(ini_r[t] == 0)
    def _():
        out_ref[...] = out_ref[...] + oe


def _grouped(sorted_x, wsorted, W1, B1, W2, B2, W3, B3, eid, blk, gs, gend,
             ini):
    grid_spec = pltpu.PrefetchScalarGridSpec(
        num_scalar_prefetch=5,
        grid=(NT,),
        in_specs=[
            pl.BlockSpec((BLK, DIM), lambda t, ei, bl, g0, g1, i0: (bl[t], 0)),
            pl.BlockSpec((BLK, 128), lambda t, ei, bl, g0, g1, i0: (bl[t], 0)),
            pl.BlockSpec((1, INTER, DIM),
                         lambda t, ei, bl, g0, g1, i0: (ei[t], 0, 0)),
            pl.BlockSpec((1, 1, INTER),
                         lambda t, ei, bl, g0, g1, i0: (ei[t], 0, 0)),
            pl.BlockSpec((1, DIM, INTER),
                         lambda t, ei, bl, g0, g1, i0: (ei[t], 0, 0)),
            pl.BlockSpec((1, 1, DIM),
                         lambda t, ei, bl, g0, g1, i0: (ei[t], 0, 0)),
            pl.BlockSpec((1, INTER, DIM),
                         lambda t, ei, bl, g0, g1, i0: (ei[t], 0, 0)),
            pl.BlockSpec((1, 1, INTER),
                         lambda t, ei, bl, g0, g1, i0: (ei[t], 0, 0)),
        ],
        out_specs=pl.BlockSpec((BLK, DIM),
                               lambda t, ei, bl, g0, g1, i0: (bl[t], 0)),
    )
    return pl.pallas_call(
        _grouped_block,
        grid_spec=grid_spec,
        out_shape=jax.ShapeDtypeStruct((TS, DIM), jnp.float32),
        compiler_params=pltpu.CompilerParams(
            dimension_semantics=("arbitrary",)),
    )(eid, blk, gs, gend, ini, sorted_x, wsorted, W1, B1[:, None], W2,
      B2[:, None], W3, B3[:, None])


def _shared_block(x_ref, w1_ref, b1_ref, w2_ref, b2_ref, w3_ref, b3_ref,
                  out_ref):
    xs = x_ref[...]
    h1 = lax.dot_general(xs, w1_ref[...], (((1,), (1,)), ((), ())),
                         preferred_element_type=jnp.float32) + b1_ref[...]
    h3 = lax.dot_general(xs, w3_ref[...], (((1,), (1,)), ((), ())),
                         preferred_element_type=jnp.float32) + b3_ref[...]
    h = _silu(h1) * h3
    out_ref[...] = lax.dot_general(h, w2_ref[...], (((1,), (1,)), ((), ())),
                                   preferred_element_type=jnp.float32
                                   ) + b2_ref[...]


def _shared(xf, SW1, SB1, SW2, SB2, SW3, SB3):
    bm = 1024
    return pl.pallas_call(
        _shared_block,
        grid=(T // bm,),
        in_specs=[
            pl.BlockSpec((bm, DIM), lambda i: (i, 0)),
            pl.BlockSpec((SINTER, DIM), lambda i: (0, 0)),
            pl.BlockSpec((1, SINTER), lambda i: (0, 0)),
            pl.BlockSpec((DIM, SINTER), lambda i: (0, 0)),
            pl.BlockSpec((1, DIM), lambda i: (0, 0)),
            pl.BlockSpec((SINTER, DIM), lambda i: (0, 0)),
            pl.BlockSpec((1, SINTER), lambda i: (0, 0)),
        ],
        out_specs=pl.BlockSpec((bm, DIM), lambda i: (i, 0)),
        out_shape=jax.ShapeDtypeStruct((T, DIM), jnp.float32),
    )(xf, SW1, SB1.reshape(1, SINTER), SW2, SB2.reshape(1, DIM),
      SW3, SB3.reshape(1, SINTER))


def _combine_body(shared_hbm, routed_hbm, dest, yhbm, idx0,
                  r0a, r0b, r1a, r1b, sa, sb,
                  sg0a, sg0b, sg1a, sg1b, sg2a, sg2b, sya, syb):
    wid = lax.axis_index("s") * 2 + lax.axis_index("c")
    base = wid * CHUNK
    EN = 16
    ESUB = TPW // EN
    r0 = [r0a, r0b]
    r1 = [r1a, r1b]
    sB = [sa, sb]
    sg0 = [sg0a, sg0b]
    sg1 = [sg1a, sg1b]
    sg2 = [sg2a, sg2b]
    sy = [sya, syb]

    pltpu.sync_copy(dest.at[pl.ds(base, CHUNK)], idx0)

    def issue(c):
        b = c & 1
        tbase = wid * TPW + EN * c
        g0 = pltpu.async_copy(
            routed_hbm.at[idx0.at[pl.ds(EN * c, EN)]], r0[b], sg0[b])
        g1 = pltpu.async_copy(
            routed_hbm.at[idx0.at[pl.ds(TPW + EN * c, EN)]], r1[b], sg1[b])
        g2 = pltpu.async_copy(shared_hbm.at[pl.ds(tbase, EN)], sB[b], sg2[b])
        return (g0, g1, g2)

    pend_y = [None, None]
    h = issue(0)
    for c in range(ESUB):
        b = c & 1
        if c + 1 < ESUB:
            nb = (c + 1) & 1
            if pend_y[nb] is not None:
                pend_y[nb].wait()
            h_next = issue(c + 1)
        h[0].wait()
        h[1].wait()
        h[2].wait()

        def body(t, carry):
            for d in range(DIM // 16):
                sl = pl.ds(16 * d, 16)
                sB[b][t, sl] = sB[b][t, sl] + r0[b][t, sl] + r1[b][t, sl]
            return carry

        lax.fori_loop(0, EN, body, 0)
        tbase = wid * TPW + EN * c
        pend_y[b] = pltpu.async_copy(sB[b], yhbm.at[pl.ds(tbase, EN)], sy[b])
        if c + 1 < ESUB:
            h = h_next
    pend_y[0].wait()
    pend_y[1].wait()


def _combine(shared, routed, dest):
    mesh = plsc.VectorSubcoreMesh(core_axis_name="c", subcore_axis_name="s")
    f = pl.kernel(
        _combine_body,
        out_type=jax.ShapeDtypeStruct((T, DIM), jnp.float32),
        mesh=mesh,
        scratch_types=(
            [pltpu.VMEM((CHUNK,), jnp.int32)]
            + [pltpu.VMEM((16, DIM), jnp.float32)] * 6
            + [pltpu.SemaphoreType.DMA] * 8
        ),
    )
    return f(shared, routed, dest)


def kernel(x, gate_w, gate_b, W1, B1, W2, B2, W3, B3, SW1, SB1, SW2, SB2, SW3, SB3):
    bsz, seq, hdim = x.shape
    xf = x.reshape(-1, hdim)
    e2d, w116, w216 = _gate(xf, gate_w, gate_b)
    eperm = e2d.reshape(NW, TPW, 2).transpose(0, 2, 1).reshape(TS)
    wperm = jnp.stack([w116.reshape(NW, TPW, 128), w216.reshape(NW, TPW, 128)],
                      axis=1).reshape(TS, 128)
    dest, counts16 = _b1(eperm)
    sorted_x, wsorted = _b2(xf, wperm, dest)
    shared = _shared(xf, SW1, SB1, SW2, SB2, SW3, SB3)
    eid, blk, gs, gend, ini = _metadata(counts16)
    routed = _grouped(sorted_x, wsorted, W1, B1, W2, B2, W3, B3, eid, blk,
                      gs, gend, ini)
    y = _combine(shared, routed, dest)
    return y.reshape(bsz, seq, hdim)
